# baseline (speedup 1.0000x reference)
import jax
import jax.numpy as jnp
from jax import lax
from jax.experimental import pallas as pl
from jax.experimental.pallas import tpu as pltpu
from jax.experimental.pallas import tpu_sc as plsc

_N = 1024
_K = 1024
_D = 64
_L = 16
_RPW = 32
_HALF = 16
_NT = 4
_KT = 8
_KV = _K // _L
_KC = 512
_DU = 4
_SU = 8


def _all_lanes_reduce(v, op):
    idx = lax.iota(jnp.int32, _L)
    dnums = lax.GatherDimensionNumbers(
        offset_dims=(), collapsed_slice_dims=(0,), start_index_map=(0,)
    )
    for sh in (8, 4, 2, 1):
        perm = jnp.bitwise_and(idx + sh, _L - 1)
        rot = lax.gather(
            v, perm[:, None], dnums, slice_sizes=(1,),
            mode=lax.GatherScatterMode.PROMISE_IN_BOUNDS,
        )
        v = op(v, rot)
    return v


def _sc_body(xsp_hbm, ct_hbm, o_hbm, xsp_v, ct_v, logit_v):
    c = lax.axis_index("c")
    s = lax.axis_index("s")
    w = s * 2 + c
    row0 = w * _RPW

    def half_body(half, carry0):
        rbase = row0 + half * _HALF
        pltpu.sync_copy(xsp_hbm.at[pl.ds(rbase, _HALF)], xsp_v)

        def kc_body(kc, carry1):
            pltpu.sync_copy(ct_hbm.at[kc], ct_v)
            kv0 = kc * (_KC // _L)

            def tile_body(t, carry2):
                nb = (t // (_KC // _L // _KT)) * _NT
                kb = (t % (_KC // _L // _KT)) * _KT

                def d_body(du, accs):
                    accs = list(accs)
                    for u in range(_DU):
                        d = du * _DU + u
                        cts = [
                            ct_v[d, pl.ds((kb + j) * _L, _L)] for j in range(_KT)
                        ]
                        for i in range(_NT):
                            xv = xsp_v[nb + i, pl.ds(d * _L, _L)]
                            for j in range(_KT):
                                accs[i * _KT + j] = accs[i * _KT + j] + jnp.abs(
                                    xv - cts[j]
                                )
                    return tuple(accs)

                init = tuple(
                    jnp.zeros((_L,), jnp.float32) for _ in range(_NT * _KT)
                )
                accs = lax.fori_loop(0, _D // _DU, d_body, init)
                for i in range(_NT):
                    for j in range(_KT):
                        a = accs[i * _KT + j]
                        logit_v[nb + i, pl.ds((kv0 + kb + j) * _L, _L)] = (
                            a * a * (-0.5)
                        )
                return carry2

            lax.fori_loop(0, (_HALF // _NT) * (_KC // _L // _KT), tile_body, 0)
            return carry1

        lax.fori_loop(0, _K // _KC, kc_body, 0)

        def srow(i, carry1):
            def mx(g, m):
                for u in range(_SU):
                    m = jnp.maximum(
                        m, logit_v[i, pl.ds((g * _SU + u) * _L, _L)]
                    )
                return m

            m = lax.fori_loop(0, _KV // _SU, mx, jnp.full((_L,), -1e30, jnp.float32))
            ms = _all_lanes_reduce(m, jnp.maximum)

            def ex(g, acc):
                for u in range(_SU):
                    kv = g * _SU + u
                    e = jnp.exp(logit_v[i, pl.ds(kv * _L, _L)] - ms)
                    logit_v[i, pl.ds(kv * _L, _L)] = e
                    acc = acc + e
                return acc

            sv = lax.fori_loop(0, _KV // _SU, ex, jnp.zeros((_L,), jnp.float32))
            inv = 1.0 / _all_lanes_reduce(sv, jnp.add)

            def dv(g, carry2):
                for u in range(_SU):
                    kv = g * _SU + u
                    logit_v[i, pl.ds(kv * _L, _L)] = (
                        logit_v[i, pl.ds(kv * _L, _L)] * inv
                    )
                return carry2

            lax.fori_loop(0, _KV // _SU, dv, 0)
            return carry1

        lax.fori_loop(0, _HALF, srow, 0)
        pltpu.sync_copy(logit_v, o_hbm.at[pl.ds(rbase, _HALF)])
        return carry0

    lax.fori_loop(0, _RPW // _HALF, half_body, 0)


def kernel(x, placeCells):
    x = jnp.reshape(x, (-1, _D))
    ct = placeCells.T
    ct4 = jnp.transpose(
        jnp.reshape(ct, (_D, _K // _KC, _KC)), (1, 0, 2)
    )
    xsp = jnp.reshape(
        jnp.broadcast_to(x[:, :, None], (_N, _D, _L)), (_N, _D * _L)
    )
    mesh = plsc.VectorSubcoreMesh(core_axis_name="c", subcore_axis_name="s")
    f = pl.kernel(
        _sc_body,
        out_type=jax.ShapeDtypeStruct((_N, _K), jnp.float32),
        mesh=mesh,
        scratch_types=[
            pltpu.VMEM((_HALF, _D * _L), jnp.float32),
            pltpu.VMEM((_D, _KC), jnp.float32),
            pltpu.VMEM((_HALF, _K), jnp.float32),
        ],
    )
    return f(xsp, ct4)


# speedup vs baseline: 2.0827x; 2.0827x over previous
import jax
import jax.numpy as jnp
from jax import lax
from jax.experimental import pallas as pl
from jax.experimental.pallas import tpu as pltpu
from jax.experimental.pallas import tpu_sc as plsc

_N = 1024
_K = 1024
_D = 64
_L = 16
_RPW = 32
_HALF = 16
_NT = 4
_KT = 8
_KV = _K // _L
_KC = 512
_DU = 1
_SU = 8


def _all_lanes_reduce(v, op):
    idx = lax.iota(jnp.int32, _L)
    dnums = lax.GatherDimensionNumbers(
        offset_dims=(), collapsed_slice_dims=(0,), start_index_map=(0,)
    )
    for sh in (8, 4, 2, 1):
        perm = jnp.bitwise_and(idx + sh, _L - 1)
        rot = lax.gather(
            v, perm[:, None], dnums, slice_sizes=(1,),
            mode=lax.GatherScatterMode.PROMISE_IN_BOUNDS,
        )
        v = op(v, rot)
    return v


def _sc_body(xsp_hbm, ct_hbm, o_hbm, xsp_v, ct_v, logit_v):
    c = lax.axis_index("c")
    s = lax.axis_index("s")
    w = s * 2 + c
    row0 = w * _RPW

    def half_body(half, carry0):
        rbase = row0 + half * _HALF
        pltpu.sync_copy(xsp_hbm.at[pl.ds(rbase, _HALF)], xsp_v)

        def kc_body(kc, carry1):
            pltpu.sync_copy(ct_hbm.at[kc], ct_v)
            kv0 = kc * (_KC // _L)

            def tile_body(t, carry2):
                nb = (t // (_KC // _L // _KT)) * _NT
                kb = (t % (_KC // _L // _KT)) * _KT

                def d_body(du, accs):
                    accs = list(accs)
                    for u in range(_DU):
                        d = du * _DU + u
                        cts = [
                            ct_v[d, pl.ds((kb + j) * _L, _L)] for j in range(_KT)
                        ]
                        for i in range(_NT):
                            xv = xsp_v[nb + i, pl.ds(d * _L, _L)]
                            for j in range(_KT):
                                accs[i * _KT + j] = accs[i * _KT + j] + jnp.abs(
                                    xv - cts[j]
                                )
                    return tuple(accs)

                init = tuple(
                    jnp.zeros((_L,), jnp.float32) for _ in range(_NT * _KT)
                )
                accs = lax.fori_loop(0, _D // _DU, d_body, init)
                for i in range(_NT):
                    for j in range(_KT):
                        a = accs[i * _KT + j]
                        logit_v[nb + i, pl.ds((kv0 + kb + j) * _L, _L)] = (
                            a * a * (-0.5)
                        )
                return carry2

            lax.fori_loop(0, (_HALF // _NT) * (_KC // _L // _KT), tile_body, 0)
            return carry1

        lax.fori_loop(0, _K // _KC, kc_body, 0)

        def srow(i, carry1):
            def mx(g, m):
                for u in range(_SU):
                    m = jnp.maximum(
                        m, logit_v[i, pl.ds((g * _SU + u) * _L, _L)]
                    )
                return m

            m = lax.fori_loop(0, _KV // _SU, mx, jnp.full((_L,), -1e30, jnp.float32))
            ms = _all_lanes_reduce(m, jnp.maximum)

            def ex(g, acc):
                for u in range(_SU):
                    kv = g * _SU + u
                    e = jnp.exp(logit_v[i, pl.ds(kv * _L, _L)] - ms)
                    logit_v[i, pl.ds(kv * _L, _L)] = e
                    acc = acc + e
                return acc

            sv = lax.fori_loop(0, _KV // _SU, ex, jnp.zeros((_L,), jnp.float32))
            inv = 1.0 / _all_lanes_reduce(sv, jnp.add)

            def dv(g, carry2):
                for u in range(_SU):
                    kv = g * _SU + u
                    logit_v[i, pl.ds(kv * _L, _L)] = (
                        logit_v[i, pl.ds(kv * _L, _L)] * inv
                    )
                return carry2

            lax.fori_loop(0, _KV // _SU, dv, 0)
            return carry1

        lax.fori_loop(0, _HALF, srow, 0)
        pltpu.sync_copy(logit_v, o_hbm.at[pl.ds(rbase, _HALF)])
        return carry0

    lax.fori_loop(0, _RPW // _HALF, half_body, 0)


def kernel(x, placeCells):
    x = jnp.reshape(x, (-1, _D))
    ct = placeCells.T
    ct4 = jnp.transpose(
        jnp.reshape(ct, (_D, _K // _KC, _KC)), (1, 0, 2)
    )
    xsp = jnp.reshape(
        jnp.broadcast_to(x[:, :, None], (_N, _D, _L)), (_N, _D * _L)
    )
    mesh = plsc.VectorSubcoreMesh(core_axis_name="c", subcore_axis_name="s")
    f = pl.kernel(
        _sc_body,
        out_type=jax.ShapeDtypeStruct((_N, _K), jnp.float32),
        mesh=mesh,
        scratch_types=[
            pltpu.VMEM((_HALF, _D * _L), jnp.float32),
            pltpu.VMEM((_D, _KC), jnp.float32),
            pltpu.VMEM((_HALF, _K), jnp.float32),
        ],
    )
    return f(xsp, ct4)


# speedup vs baseline: 6.0658x; 2.9125x over previous
import jax
import jax.numpy as jnp
from jax import lax
from jax.experimental import pallas as pl
from jax.experimental.pallas import tpu as pltpu
from jax.experimental.pallas import tpu_sc as plsc

_N = 1024
_K = 1024
_D = 64

_L = 16
_S = 128
_W = 32
_RPW = _S // _W
_NT = min(4, _RPW)
_KT = 8
_KV = _K // _L
_KC = 512
_SU = 8

_NB = 128


def _all_lanes_reduce(v, op):
    idx = lax.iota(jnp.int32, _L)
    dnums = lax.GatherDimensionNumbers(
        offset_dims=(), collapsed_slice_dims=(0,), start_index_map=(0,)
    )
    for sh in (8, 4, 2, 1):
        perm = jnp.bitwise_and(idx + sh, _L - 1)
        rot = lax.gather(
            v, perm[:, None], dnums, slice_sizes=(1,),
            mode=lax.GatherScatterMode.PROMISE_IN_BOUNDS,
        )
        v = op(v, rot)
    return v


def _sc_body(xsp_hbm, ct_hbm, o_hbm, xsp_v, ct_v, logit_v):
    c = lax.axis_index("c")
    s = lax.axis_index("s")
    w = s * 2 + c
    rbase = w * _RPW
    pltpu.sync_copy(xsp_hbm.at[pl.ds(rbase, _RPW)], xsp_v)

    def kc_body(kc, carry1):
        pltpu.sync_copy(ct_hbm.at[kc], ct_v)
        kv0 = kc * (_KC // _L)

        def tile_body(t, carry2):
            nb = (t // (_KC // _L // _KT)) * _NT
            kb = (t % (_KC // _L // _KT)) * _KT

            def d_body(d, accs):
                accs = list(accs)
                cts = [ct_v[d, pl.ds((kb + j) * _L, _L)] for j in range(_KT)]
                for i in range(_NT):
                    xv = xsp_v[nb + i, pl.ds(d * _L, _L)]
                    for j in range(_KT):
                        accs[i * _KT + j] = accs[i * _KT + j] + jnp.abs(
                            xv - cts[j]
                        )
                return tuple(accs)

            init = tuple(jnp.zeros((_L,), jnp.float32) for _ in range(_NT * _KT))
            accs = lax.fori_loop(0, _D, d_body, init)
            for i in range(_NT):
                for j in range(_KT):
                    a = accs[i * _KT + j]
                    logit_v[nb + i, pl.ds((kv0 + kb + j) * _L, _L)] = (
                        a * a * (-0.5)
                    )
            return carry2

        lax.fori_loop(0, (_RPW // _NT) * (_KC // _L // _KT), tile_body, 0)
        return carry1

    lax.fori_loop(0, _K // _KC, kc_body, 0)

    def srow(i, carry1):
        def mx(g, m):
            for u in range(_SU):
                m = jnp.maximum(m, logit_v[i, pl.ds((g * _SU + u) * _L, _L)])
            return m

        m = lax.fori_loop(0, _KV // _SU, mx, jnp.full((_L,), -1e30, jnp.float32))
        ms = _all_lanes_reduce(m, jnp.maximum)

        def ex(g, acc):
            for u in range(_SU):
                kv = g * _SU + u
                e = jnp.exp(logit_v[i, pl.ds(kv * _L, _L)] - ms)
                logit_v[i, pl.ds(kv * _L, _L)] = e
                acc = acc + e
            return acc

        sv = lax.fori_loop(0, _KV // _SU, ex, jnp.zeros((_L,), jnp.float32))
        inv = 1.0 / _all_lanes_reduce(sv, jnp.add)

        def dv(g, carry2):
            for u in range(_SU):
                kv = g * _SU + u
                logit_v[i, pl.ds(kv * _L, _L)] = (
                    logit_v[i, pl.ds(kv * _L, _L)] * inv
                )
            return carry2

        lax.fori_loop(0, _KV // _SU, dv, 0)
        return carry1

    lax.fori_loop(0, _RPW, srow, 0)
    pltpu.sync_copy(logit_v, o_hbm.at[pl.ds(rbase, _RPW)])


def _sc_part(x_sc, placeCells):
    ct = placeCells.T
    ct4 = jnp.transpose(
        jnp.reshape(ct, (_D, _K // _KC, _KC)), (1, 0, 2)
    )
    xsp = jnp.reshape(
        jnp.broadcast_to(x_sc[:, :, None], (_S, _D, _L)), (_S, _D * _L)
    )
    mesh = plsc.VectorSubcoreMesh(core_axis_name="c", subcore_axis_name="s")
    f = pl.kernel(
        _sc_body,
        out_type=jax.ShapeDtypeStruct((_S, _K), jnp.float32),
        mesh=mesh,
        scratch_types=[
            pltpu.VMEM((_RPW, _D * _L), jnp.float32),
            pltpu.VMEM((_D, _KC), jnp.float32),
            pltpu.VMEM((_RPW, _K), jnp.float32),
        ],
    )
    return f(xsp, ct4)


def _tc_body(x_ref, ct_ref, o_ref):
    xb = x_ref[:]
    ct = ct_ref[:]
    acc = jnp.zeros((_NB, _K), jnp.float32)
    for d in range(_D):
        acc = acc + jnp.abs(xb[:, d][:, None] - ct[d, :][None, :])
    logits = acc * acc * (-0.5)
    m = jnp.max(logits, axis=1, keepdims=True)
    e = jnp.exp(logits - m)
    s = jnp.sum(e, axis=1, keepdims=True)
    o_ref[:] = e / s


def _tc_part(x_tc, placeCells):
    n = x_tc.shape[0]
    ct = placeCells.T
    return pl.pallas_call(
        _tc_body,
        grid=(n // _NB,),
        in_specs=[
            pl.BlockSpec((_NB, _D), lambda i: (i, 0)),
            pl.BlockSpec((_D, _K), lambda i: (0, 0)),
        ],
        out_specs=pl.BlockSpec((_NB, _K), lambda i: (i, 0)),
        out_shape=jax.ShapeDtypeStruct((n, _K), jnp.float32),
    )(x_tc, ct)


def kernel(x, placeCells):
    x = jnp.reshape(x, (-1, _D))
    out_tc = _tc_part(x[: _N - _S], placeCells)
    out_sc = _sc_part(x[_N - _S :], placeCells)
    return jnp.concatenate([out_tc, out_sc], axis=0)


# speedup vs baseline: 9.7421x; 1.6061x over previous
import jax
import jax.numpy as jnp
from jax.experimental import pallas as pl
from jax.experimental.pallas import tpu as pltpu

_N = 1024
_K = 1024
_D = 64
_NB = 256


def _body(x_ref, ct_ref, o_ref):
    xb = x_ref[:]
    ct = ct_ref[:]
    acc = jnp.zeros((_NB, _K), jnp.float32)
    for d in range(_D):
        acc = acc + jnp.abs(xb[:, d][:, None] - ct[d, :][None, :])
    logits = acc * acc * (-0.5)
    m = jnp.max(logits, axis=1, keepdims=True)
    e = jnp.exp(logits - m)
    s = jnp.sum(e, axis=1, keepdims=True)
    o_ref[:] = e / s


def kernel(x, placeCells):
    x = jnp.reshape(x, (-1, _D))
    ct = placeCells.T
    return pl.pallas_call(
        _body,
        grid=(_N // _NB,),
        in_specs=[
            pl.BlockSpec((_NB, _D), lambda i: (i, 0)),
            pl.BlockSpec((_D, _K), lambda i: (0, 0)),
        ],
        out_specs=pl.BlockSpec((_NB, _K), lambda i: (i, 0)),
        out_shape=jax.ShapeDtypeStruct((_N, _K), jnp.float32),
    )(x, ct)
